# Initial kernel scaffold; baseline (speedup 1.0000x reference)
import jax
import jax.numpy as jnp
from jax.experimental import pallas as pl


def kernel(node_features, edge_index, deg, numBBs, W, b, gamma_bb, beta_bb, gamma_rel, beta_rel):
    raise NotImplementedError("write your pallas kernel here")


# speedup vs baseline: 2.9112x; 2.9112x over previous
import functools

import jax
import jax.numpy as jnp
from jax import lax
from jax.experimental import pallas as pl
from jax.experimental.pallas import tpu as pltpu
from jax.experimental.pallas import tpu_sc as plsc

N = 10000
E = 160000
D = 256
H = D // 2
GROUPS = 4
GSZ = D // GROUPS
ROWS_BLK = 400
NBLK = N // ROWS_BLK
CHUNK = 128
N_SUBCORES = 16
ROWS_PER_SUB = 640
N_ACC = ROWS_PER_SUB * N_SUBCORES
E_PAD = ((E + CHUNK * N_SUBCORES - 1) // (CHUNK * N_SUBCORES)) * (CHUNK * N_SUBCORES)
CHUNKS_PER_SUB = E_PAD // CHUNK // N_SUBCORES


def _tcpre_body(nb_ref, x_ref, w_ref, gbb_ref, bbb_ref, grel_ref, brel_ref, o_ref):
    i = pl.program_id(0)
    x = x_ref[...]
    parts = []
    for g in range(GROUPS):
        xg = x[:, GSZ * g:GSZ * (g + 1)]
        m = jnp.mean(xg, axis=1, keepdims=True)
        v = jnp.mean((xg - m) ** 2, axis=1, keepdims=True)
        parts.append((xg - m) * lax.rsqrt(v + 1e-5))
    xn = jnp.concatenate(parts, axis=1)
    rows = ROWS_BLK * i + lax.broadcasted_iota(jnp.int32, (ROWS_BLK, 1), 0)
    is_bb = rows < nb_ref[0, 0]
    gamma = jnp.where(is_bb, gbb_ref[...], grel_ref[...])
    beta = jnp.where(is_bb, bbb_ref[...], brel_ref[...])
    xa = jnp.maximum(xn * gamma + beta, 0.0)
    o_ref[...] = jnp.dot(xa, w_ref[...], preferred_element_type=jnp.float32)


def _tc_support(nb, x, w, gbb, bbb, grel, brel):
    return pl.pallas_call(
        _tcpre_body,
        grid=(NBLK, 2),
        in_specs=[
            pl.BlockSpec(memory_space=pltpu.SMEM),
            pl.BlockSpec((ROWS_BLK, D), lambda i, h: (i, 0)),
            pl.BlockSpec((D, H), lambda i, h: (0, h)),
            pl.BlockSpec((1, D), lambda i, h: (0, 0)),
            pl.BlockSpec((1, D), lambda i, h: (0, 0)),
            pl.BlockSpec((1, D), lambda i, h: (0, 0)),
            pl.BlockSpec((1, D), lambda i, h: (0, 0)),
        ],
        out_specs=pl.BlockSpec((ROWS_BLK, H), lambda i, h: (h * NBLK + i, 0)),
        out_shape=jax.ShapeDtypeStruct((2 * N, H), jnp.float32),
    )(nb, x, w, gbb, bbb, grel, brel)


def _sc_body(sup_hbm, col2_hbm, row_hbm, zeros_hbm, out_hbm,
             colv, rowv, gbuf, acc, sem):
    c = lax.axis_index("c")
    s = lax.axis_index("s")
    base = ROWS_PER_SUB * s
    pltpu.sync_copy(zeros_hbm, acc.at[pl.ds(base, ROWS_PER_SUB)])
    plsc.subcore_barrier()

    def body(k, carry):
        off = CHUNK * (s + N_SUBCORES * k)
        pltpu.sync_copy(col2_hbm.at[c, pl.ds(off, CHUNK)], colv)
        pltpu.sync_copy(row_hbm.at[pl.ds(off, CHUNK)], rowv)
        pltpu.async_copy(sup_hbm.at[colv], gbuf, sem).wait()
        pltpu.sync_copy(gbuf, acc.at[rowv], add=True)
        return carry

    lax.fori_loop(0, CHUNKS_PER_SUB, body, 0)
    plsc.subcore_barrier()
    pltpu.sync_copy(acc.at[pl.ds(base, ROWS_PER_SUB)],
                    out_hbm.at[c, pl.ds(base, ROWS_PER_SUB)])


@functools.cache
def _sc_segsum():
    return pl.kernel(
        _sc_body,
        out_type=jax.ShapeDtypeStruct((2, N_ACC, H), jnp.float32),
        mesh=plsc.VectorSubcoreMesh(core_axis_name="c", subcore_axis_name="s"),
        scratch_types=[
            pltpu.VMEM((CHUNK,), jnp.int32),
            pltpu.VMEM((CHUNK,), jnp.int32),
            pltpu.VMEM((CHUNK, H), jnp.float32),
            pltpu.VMEM_SHARED((N_ACC, H), jnp.float32),
            pltpu.SemaphoreType.DMA,
        ],
    )


def _tcpost_body(raw_ref, deg_ref, b_ref, o_ref):
    cat = jnp.concatenate([raw_ref[0], raw_ref[1]], axis=1)
    o_ref[...] = cat / deg_ref[...] + b_ref[...]


def _tc_post(raw, deg, b):
    return pl.pallas_call(
        _tcpost_body,
        grid=(NBLK,),
        in_specs=[
            pl.BlockSpec((2, ROWS_BLK, H), lambda i: (0, i, 0)),
            pl.BlockSpec((ROWS_BLK, 1), lambda i: (i, 0)),
            pl.BlockSpec((1, D), lambda i: (0, 0)),
        ],
        out_specs=pl.BlockSpec((ROWS_BLK, D), lambda i: (i, 0)),
        out_shape=jax.ShapeDtypeStruct((N, D), jnp.float32),
    )(raw, deg, b)


def kernel(node_features, edge_index, deg, numBBs, W, b,
           gamma_bb, beta_bb, gamma_rel, beta_rel):
    nb = jnp.asarray(numBBs, jnp.int32).reshape(1, 1)
    sup = _tc_support(nb, node_features, W,
                      gamma_bb.reshape(1, D), beta_bb.reshape(1, D),
                      gamma_rel.reshape(1, D), beta_rel.reshape(1, D))
    row = edge_index[0]
    col = edge_index[1]
    pad = E_PAD - E
    rowp = jnp.concatenate([row, jnp.full((pad,), N, jnp.int32)])
    colp = jnp.concatenate([col, jnp.zeros((pad,), jnp.int32)])
    col2 = jnp.stack([colp, colp + N])
    zeros = jnp.zeros((ROWS_PER_SUB, H), jnp.float32)
    raw = _sc_segsum()(sup, col2, rowp, zeros)
    return _tc_post(raw, deg.reshape(N, 1), b.reshape(1, D))


# speedup vs baseline: 3.0251x; 1.0391x over previous
import functools

import jax
import jax.numpy as jnp
from jax import lax
from jax.experimental import pallas as pl
from jax.experimental.pallas import tpu as pltpu
from jax.experimental.pallas import tpu_sc as plsc

N = 10000
E = 160000
D = 256
H = D // 2
GROUPS = 4
GSZ = D // GROUPS
ROWS_BLK = 400
NBLK = N // ROWS_BLK
CHUNK = 128
N_SUBCORES = 16
ROWS_PER_SUB = 640
N_ACC = ROWS_PER_SUB * N_SUBCORES
NBUF = 2
NROUND = 40
CHUNKS_PER_SUB = NBUF * NROUND
E_PAD = CHUNK * N_SUBCORES * CHUNKS_PER_SUB


def _tcpre_body(nb_ref, x_ref, w_ref, gbb_ref, bbb_ref, grel_ref, brel_ref, o_ref):
    i = pl.program_id(0)
    x = x_ref[...]
    parts = []
    for g in range(GROUPS):
        xg = x[:, GSZ * g:GSZ * (g + 1)]
        m = jnp.mean(xg, axis=1, keepdims=True)
        v = jnp.mean((xg - m) ** 2, axis=1, keepdims=True)
        parts.append((xg - m) * lax.rsqrt(v + 1e-5))
    xn = jnp.concatenate(parts, axis=1)
    rows = ROWS_BLK * i + lax.broadcasted_iota(jnp.int32, (ROWS_BLK, 1), 0)
    is_bb = rows < nb_ref[0, 0]
    gamma = jnp.where(is_bb, gbb_ref[...], grel_ref[...])
    beta = jnp.where(is_bb, bbb_ref[...], brel_ref[...])
    xa = jnp.maximum(xn * gamma + beta, 0.0)
    o_ref[...] = jnp.dot(xa, w_ref[...], preferred_element_type=jnp.float32)


def _tc_support(nb, x, w, gbb, bbb, grel, brel):
    return pl.pallas_call(
        _tcpre_body,
        grid=(NBLK, 2),
        in_specs=[
            pl.BlockSpec(memory_space=pltpu.SMEM),
            pl.BlockSpec((ROWS_BLK, D), lambda i, h: (i, 0)),
            pl.BlockSpec((D, H), lambda i, h: (0, h)),
            pl.BlockSpec((1, D), lambda i, h: (0, 0)),
            pl.BlockSpec((1, D), lambda i, h: (0, 0)),
            pl.BlockSpec((1, D), lambda i, h: (0, 0)),
            pl.BlockSpec((1, D), lambda i, h: (0, 0)),
        ],
        out_specs=pl.BlockSpec((ROWS_BLK, H), lambda i, h: (h * NBLK + i, 0)),
        out_shape=jax.ShapeDtypeStruct((2 * N, H), jnp.float32),
    )(nb, x, w, gbb, bbb, grel, brel)


def _sc_body(sup_hbm, col2_hbm, row_hbm, zeros_hbm, out_hbm,
             colv, rowv, gbuf, acc, semi, semg, sems):
    c = lax.axis_index("c")
    s = lax.axis_index("s")
    base = ROWS_PER_SUB * s
    pltpu.sync_copy(zeros_hbm, acc.at[pl.ds(base, ROWS_PER_SUB)])
    plsc.subcore_barrier()

    def start_idx(k, slot):
        off = CHUNK * (s + N_SUBCORES * k)
        pltpu.async_copy(col2_hbm.at[c, pl.ds(off, CHUNK)], colv.at[slot],
                         semi.at[slot])
        pltpu.async_copy(row_hbm.at[pl.ds(off, CHUNK)], rowv.at[slot],
                         semi.at[slot])

    def wait_idx(slot):
        pltpu.make_async_copy(col2_hbm.at[c, pl.ds(0, CHUNK)], colv.at[slot],
                              semi.at[slot]).wait()
        pltpu.make_async_copy(row_hbm.at[pl.ds(0, CHUNK)], rowv.at[slot],
                              semi.at[slot]).wait()

    def start_gather(b, slot):
        pltpu.async_copy(sup_hbm.at[colv.at[slot]], gbuf.at[b], semg.at[b])

    def wait_gather(b):
        pltpu.make_async_copy(sup_hbm.at[pl.ds(0, CHUNK)], gbuf.at[b],
                              semg.at[b]).wait()

    def start_scatter(b, slot):
        pltpu.async_copy(gbuf.at[b], acc.at[rowv.at[slot]], sems.at[b],
                         add=True)

    def wait_scatter(b):
        pltpu.make_async_copy(sup_hbm.at[pl.ds(0, CHUNK)], gbuf.at[b],
                              sems.at[b]).wait()

    def emit_round(g, g2, parity):
        pslot = parity * NBUF
        nslot = (1 - parity) * NBUF
        for b in range(NBUF):
            if parity == 0:
                @pl.when(g2 > 0)
                def _():
                    wait_scatter(b)
            else:
                wait_scatter(b)
            wait_idx(pslot + b)
            start_gather(b, pslot + b)
        if parity == 0:
            @pl.when(g2 > 0)
            def _():
                for b in range(NBUF):
                    start_idx((g + 1) * NBUF + b, nslot + b)
        else:
            @pl.when(g2 < NROUND // 2 - 1)
            def _():
                for b in range(NBUF):
                    start_idx((g + 1) * NBUF + b, nslot + b)
        for b in range(NBUF):
            wait_gather(b)
            start_scatter(b, pslot + b)

    for b in range(NBUF):
        start_idx(b, b)
    for b in range(NBUF):
        start_idx(NBUF + b, NBUF + b)

    def outer(g2, carry):
        emit_round(2 * g2, g2, 0)
        emit_round(2 * g2 + 1, g2, 1)
        return carry

    lax.fori_loop(0, NROUND // 2, outer, 0)
    for b in range(NBUF):
        wait_scatter(b)
    plsc.subcore_barrier()
    pltpu.sync_copy(acc.at[pl.ds(base, ROWS_PER_SUB)],
                    out_hbm.at[c, pl.ds(base, ROWS_PER_SUB)])


@functools.cache
def _sc_segsum():
    return pl.kernel(
        _sc_body,
        out_type=jax.ShapeDtypeStruct((2, N_ACC, H), jnp.float32),
        mesh=plsc.VectorSubcoreMesh(core_axis_name="c", subcore_axis_name="s"),
        scratch_types=[
            pltpu.VMEM((2 * NBUF, CHUNK), jnp.int32),
            pltpu.VMEM((2 * NBUF, CHUNK), jnp.int32),
            pltpu.VMEM((NBUF, CHUNK, H), jnp.float32),
            pltpu.VMEM_SHARED((N_ACC, H), jnp.float32),
            pltpu.SemaphoreType.DMA((2 * NBUF,)),
            pltpu.SemaphoreType.DMA((NBUF,)),
            pltpu.SemaphoreType.DMA((NBUF,)),
        ],
    )


def _tcpost_body(raw_ref, deg_ref, b_ref, o_ref):
    cat = jnp.concatenate([raw_ref[0], raw_ref[1]], axis=1)
    o_ref[...] = cat / deg_ref[...] + b_ref[...]


def _tc_post(raw, deg, b):
    return pl.pallas_call(
        _tcpost_body,
        grid=(NBLK,),
        in_specs=[
            pl.BlockSpec((2, ROWS_BLK, H), lambda i: (0, i, 0)),
            pl.BlockSpec((ROWS_BLK, 1), lambda i: (i, 0)),
            pl.BlockSpec((1, D), lambda i: (0, 0)),
        ],
        out_specs=pl.BlockSpec((ROWS_BLK, D), lambda i: (i, 0)),
        out_shape=jax.ShapeDtypeStruct((N, D), jnp.float32),
    )(raw, deg, b)


def kernel(node_features, edge_index, deg, numBBs, W, b,
           gamma_bb, beta_bb, gamma_rel, beta_rel):
    nb = jnp.asarray(numBBs, jnp.int32).reshape(1, 1)
    sup = _tc_support(nb, node_features, W,
                      gamma_bb.reshape(1, D), beta_bb.reshape(1, D),
                      gamma_rel.reshape(1, D), beta_rel.reshape(1, D))
    row = edge_index[0]
    col = edge_index[1]
    pad = E_PAD - E
    rowp = jnp.concatenate([row, jnp.full((pad,), N, jnp.int32)])
    colp = jnp.concatenate([col, jnp.zeros((pad,), jnp.int32)])
    col2 = jnp.stack([colp, colp + N])
    zeros = jnp.zeros((ROWS_PER_SUB, H), jnp.float32)
    raw = _sc_segsum()(sup, col2, rowp, zeros)
    return _tc_post(raw, deg.reshape(N, 1), b.reshape(1, D))


# speedup vs baseline: 3.2766x; 1.0831x over previous
import functools

import jax
import jax.numpy as jnp
from jax import lax
from jax.experimental import pallas as pl
from jax.experimental.pallas import tpu as pltpu
from jax.experimental.pallas import tpu_sc as plsc

N = 10000
E = 160000
D = 256
H = D // 2
GROUPS = 4
GSZ = D // GROUPS
ROWS_BLK = 400
NBLK = N // ROWS_BLK
CHUNK = 64
N_SUBCORES = 16
ROWS_PER_SUB = 640
N_ACC = ROWS_PER_SUB * N_SUBCORES
_DIAG = ""
NBUF = 4
NROUND = 40
CHUNKS_PER_SUB = NBUF * NROUND
E_PAD = CHUNK * N_SUBCORES * CHUNKS_PER_SUB


def _tcpre_body(nb_ref, x_ref, w_ref, gbb_ref, bbb_ref, grel_ref, brel_ref, o_ref):
    i = pl.program_id(0)
    x = x_ref[...]
    parts = []
    for g in range(GROUPS):
        xg = x[:, GSZ * g:GSZ * (g + 1)]
        m = jnp.mean(xg, axis=1, keepdims=True)
        v = jnp.mean((xg - m) ** 2, axis=1, keepdims=True)
        parts.append((xg - m) * lax.rsqrt(v + 1e-5))
    xn = jnp.concatenate(parts, axis=1)
    rows = ROWS_BLK * i + lax.broadcasted_iota(jnp.int32, (ROWS_BLK, 1), 0)
    is_bb = rows < nb_ref[0, 0]
    gamma = jnp.where(is_bb, gbb_ref[...], grel_ref[...])
    beta = jnp.where(is_bb, bbb_ref[...], brel_ref[...])
    xa = jnp.maximum(xn * gamma + beta, 0.0)
    o_ref[...] = jnp.dot(xa, w_ref[...], preferred_element_type=jnp.float32)


def _tc_support(nb, x, w, gbb, bbb, grel, brel):
    return pl.pallas_call(
        _tcpre_body,
        grid=(NBLK, 2),
        in_specs=[
            pl.BlockSpec(memory_space=pltpu.SMEM),
            pl.BlockSpec((ROWS_BLK, D), lambda i, h: (i, 0)),
            pl.BlockSpec((D, H), lambda i, h: (0, h)),
            pl.BlockSpec((1, D), lambda i, h: (0, 0)),
            pl.BlockSpec((1, D), lambda i, h: (0, 0)),
            pl.BlockSpec((1, D), lambda i, h: (0, 0)),
            pl.BlockSpec((1, D), lambda i, h: (0, 0)),
        ],
        out_specs=pl.BlockSpec((ROWS_BLK, H), lambda i, h: (h * NBLK + i, 0)),
        out_shape=jax.ShapeDtypeStruct((2 * N, H), jnp.float32),
    )(nb, x, w, gbb, bbb, grel, brel)


def _sc_body(sup_hbm, col2_hbm, row_hbm, zeros_hbm, out_hbm,
             colv, rowv, gbuf, acc, semi, semg, sems):
    c = lax.axis_index("c")
    s = lax.axis_index("s")
    base = ROWS_PER_SUB * s
    pltpu.sync_copy(zeros_hbm, acc.at[pl.ds(base, ROWS_PER_SUB)])
    plsc.subcore_barrier()

    def start_idx(k, slot):
        off = CHUNK * (s + N_SUBCORES * k)
        pltpu.async_copy(col2_hbm.at[c, pl.ds(off, CHUNK)], colv.at[slot],
                         semi.at[slot])
        pltpu.async_copy(row_hbm.at[pl.ds(off, CHUNK)], rowv.at[slot],
                         semi.at[slot])

    def wait_idx(slot):
        pltpu.make_async_copy(col2_hbm.at[c, pl.ds(0, CHUNK)], colv.at[slot],
                              semi.at[slot]).wait()
        pltpu.make_async_copy(row_hbm.at[pl.ds(0, CHUNK)], rowv.at[slot],
                              semi.at[slot]).wait()

    def start_gather(b, slot):
        if _DIAG != "nogather":
            pltpu.async_copy(sup_hbm.at[colv.at[slot]], gbuf.at[b], semg.at[b])

    def wait_gather(b):
        if _DIAG != "nogather":
            pltpu.make_async_copy(sup_hbm.at[pl.ds(0, CHUNK)], gbuf.at[b],
                                  semg.at[b]).wait()

    def start_scatter(b, slot):
        if _DIAG != "noscatter":
            pltpu.async_copy(gbuf.at[b], acc.at[rowv.at[slot]], sems.at[b],
                             add=True)

    def wait_scatter(b):
        if _DIAG != "noscatter":
            pltpu.make_async_copy(sup_hbm.at[pl.ds(0, CHUNK)], gbuf.at[b],
                                  sems.at[b]).wait()

    def emit_round(g, g2, parity):
        pslot = parity * NBUF
        nslot = (1 - parity) * NBUF
        for b in range(NBUF):
            if parity == 0:
                @pl.when(g2 > 0)
                def _():
                    wait_scatter(b)
            else:
                wait_scatter(b)
            wait_idx(pslot + b)
            start_gather(b, pslot + b)
        if parity == 0:
            @pl.when(g2 > 0)
            def _():
                for b in range(NBUF):
                    start_idx((g + 1) * NBUF + b, nslot + b)
        else:
            @pl.when(g2 < NROUND // 2 - 1)
            def _():
                for b in range(NBUF):
                    start_idx((g + 1) * NBUF + b, nslot + b)
        for b in range(NBUF):
            wait_gather(b)
            start_scatter(b, pslot + b)

    for b in range(NBUF):
        start_idx(b, b)
    for b in range(NBUF):
        start_idx(NBUF + b, NBUF + b)

    def outer(g2, carry):
        emit_round(2 * g2, g2, 0)
        emit_round(2 * g2 + 1, g2, 1)
        return carry

    lax.fori_loop(0, NROUND // 2, outer, 0)
    for b in range(NBUF):
        wait_scatter(b)
    plsc.subcore_barrier()
    pltpu.sync_copy(acc.at[pl.ds(base, ROWS_PER_SUB)],
                    out_hbm.at[c, pl.ds(base, ROWS_PER_SUB)])


@functools.cache
def _sc_segsum():
    return pl.kernel(
        _sc_body,
        out_type=jax.ShapeDtypeStruct((2, N_ACC, H), jnp.float32),
        mesh=plsc.VectorSubcoreMesh(core_axis_name="c", subcore_axis_name="s"),
        scratch_types=[
            pltpu.VMEM((2 * NBUF, CHUNK), jnp.int32),
            pltpu.VMEM((2 * NBUF, CHUNK), jnp.int32),
            pltpu.VMEM((NBUF, CHUNK, H), jnp.float32),
            pltpu.VMEM_SHARED((N_ACC, H), jnp.float32),
            pltpu.SemaphoreType.DMA((2 * NBUF,)),
            pltpu.SemaphoreType.DMA((NBUF,)),
            pltpu.SemaphoreType.DMA((NBUF,)),
        ],
    )


def _tcpost_body(raw_ref, deg_ref, b_ref, o_ref):
    cat = jnp.concatenate([raw_ref[0], raw_ref[1]], axis=1)
    o_ref[...] = cat / deg_ref[...] + b_ref[...]


def _tc_post(raw, deg, b):
    return pl.pallas_call(
        _tcpost_body,
        grid=(NBLK,),
        in_specs=[
            pl.BlockSpec((2, ROWS_BLK, H), lambda i: (0, i, 0)),
            pl.BlockSpec((ROWS_BLK, 1), lambda i: (i, 0)),
            pl.BlockSpec((1, D), lambda i: (0, 0)),
        ],
        out_specs=pl.BlockSpec((ROWS_BLK, D), lambda i: (i, 0)),
        out_shape=jax.ShapeDtypeStruct((N, D), jnp.float32),
    )(raw, deg, b)


def kernel(node_features, edge_index, deg, numBBs, W, b,
           gamma_bb, beta_bb, gamma_rel, beta_rel):
    nb = jnp.asarray(numBBs, jnp.int32).reshape(1, 1)
    sup = _tc_support(nb, node_features, W,
                      gamma_bb.reshape(1, D), beta_bb.reshape(1, D),
                      gamma_rel.reshape(1, D), beta_rel.reshape(1, D))
    row = edge_index[0]
    col = edge_index[1]
    pad = E_PAD - E
    rowp = jnp.concatenate([row, jnp.full((pad,), N, jnp.int32)])
    colp = jnp.concatenate([col, jnp.zeros((pad,), jnp.int32)])
    if _DIAG == "seqgather":
        colp = jnp.arange(E_PAD, dtype=jnp.int32) % N
    col2 = jnp.stack([colp, colp + N])
    zeros = jnp.zeros((ROWS_PER_SUB, H), jnp.float32)
    raw = _sc_segsum()(sup, col2, rowp, zeros)
    return _tc_post(raw, deg.reshape(N, 1), b.reshape(1, D))


# speedup vs baseline: 3.9555x; 1.2072x over previous
import functools

import jax
import jax.numpy as jnp
from jax import lax
from jax.experimental import pallas as pl
from jax.experimental.pallas import tpu as pltpu
from jax.experimental.pallas import tpu_sc as plsc

N = 10000
E = 160000
D = 256
H = D // 2
GROUPS = 4
GSZ = D // GROUPS
ROWS_BLK = 400
NBLK = N // ROWS_BLK
RB_PRE = 1000
NBLK_PRE = N // RB_PRE
CHUNK = 64
N_SUBCORES = 16
ROWS_PER_SUB = 640
N_ACC = ROWS_PER_SUB * N_SUBCORES
_DIAG = ""
NBUF = 4
NROUND = 40
CHUNKS_PER_SUB = NBUF * NROUND
E_PAD = CHUNK * N_SUBCORES * CHUNKS_PER_SUB


def _tcpre_body(nb_ref, x_ref, w_ref, gm_ref, gbb_ref, bbb_ref, grel_ref,
                brel_ref, o_ref):
    i = pl.program_id(0)
    x = x_ref[...]
    gm = gm_ref[...]
    m = jnp.dot(x, gm, preferred_element_type=jnp.float32)
    ex2 = jnp.dot(x * x, gm, preferred_element_type=jnp.float32)
    xn = (x - m) * lax.rsqrt(ex2 - m * m + 1e-5)
    rows = RB_PRE * i + lax.broadcasted_iota(jnp.int32, (RB_PRE, 1), 0)
    is_bb = rows < nb_ref[0, 0]
    gamma = jnp.where(is_bb, gbb_ref[...], grel_ref[...])
    beta = jnp.where(is_bb, bbb_ref[...], brel_ref[...])
    xa = jnp.maximum(xn * gamma + beta, 0.0)
    o = jnp.dot(xa, w_ref[...], preferred_element_type=jnp.float32)
    o_ref[0] = o[:, :H]
    o_ref[1] = o[:, H:]


def _tc_support(nb, x, w, gm, gbb, bbb, grel, brel):
    return pl.pallas_call(
        _tcpre_body,
        grid=(NBLK_PRE,),
        in_specs=[
            pl.BlockSpec(memory_space=pltpu.SMEM),
            pl.BlockSpec((RB_PRE, D), lambda i: (i, 0)),
            pl.BlockSpec((D, D), lambda i: (0, 0)),
            pl.BlockSpec((D, D), lambda i: (0, 0)),
            pl.BlockSpec((1, D), lambda i: (0, 0)),
            pl.BlockSpec((1, D), lambda i: (0, 0)),
            pl.BlockSpec((1, D), lambda i: (0, 0)),
            pl.BlockSpec((1, D), lambda i: (0, 0)),
        ],
        out_specs=pl.BlockSpec((2, RB_PRE, H), lambda i: (0, i, 0)),
        out_shape=jax.ShapeDtypeStruct((2, N, H), jnp.float32),
    )(nb, x, w, gm, gbb, bbb, grel, brel)


def _sc_body(sup_hbm, col2_hbm, row_hbm, zeros_hbm, out_hbm,
             colv, rowv, gbuf, acc, semi, semg, sems):
    c = lax.axis_index("c")
    s = lax.axis_index("s")
    base = ROWS_PER_SUB * s
    pltpu.sync_copy(zeros_hbm, acc.at[pl.ds(base, ROWS_PER_SUB)])
    plsc.subcore_barrier()

    def start_idx(k, slot):
        off = CHUNK * (s + N_SUBCORES * k)
        pltpu.async_copy(col2_hbm.at[c, pl.ds(off, CHUNK)], colv.at[slot],
                         semi.at[slot])
        pltpu.async_copy(row_hbm.at[pl.ds(off, CHUNK)], rowv.at[slot],
                         semi.at[slot])

    def wait_idx(slot):
        pltpu.make_async_copy(col2_hbm.at[c, pl.ds(0, CHUNK)], colv.at[slot],
                              semi.at[slot]).wait()
        pltpu.make_async_copy(row_hbm.at[pl.ds(0, CHUNK)], rowv.at[slot],
                              semi.at[slot]).wait()

    def start_gather(b, slot):
        if _DIAG != "nogather":
            pltpu.async_copy(sup_hbm.at[colv.at[slot]], gbuf.at[b], semg.at[b])

    def wait_gather(b):
        if _DIAG != "nogather":
            pltpu.make_async_copy(sup_hbm.at[pl.ds(0, CHUNK)], gbuf.at[b],
                                  semg.at[b]).wait()

    def start_scatter(b, slot):
        if _DIAG != "noscatter":
            pltpu.async_copy(gbuf.at[b], acc.at[rowv.at[slot]], sems.at[b],
                             add=True)

    def wait_scatter(b):
        if _DIAG != "noscatter":
            pltpu.make_async_copy(sup_hbm.at[pl.ds(0, CHUNK)], gbuf.at[b],
                                  sems.at[b]).wait()

    def emit_round(g, g2, parity):
        pslot = parity * NBUF
        nslot = (1 - parity) * NBUF
        for b in range(NBUF):
            if parity == 0:
                @pl.when(g2 > 0)
                def _():
                    wait_scatter(b)
            else:
                wait_scatter(b)
            wait_idx(pslot + b)
            start_gather(b, pslot + b)
        if parity == 0:
            @pl.when(g2 > 0)
            def _():
                for b in range(NBUF):
                    start_idx((g + 1) * NBUF + b, nslot + b)
        else:
            @pl.when(g2 < NROUND // 2 - 1)
            def _():
                for b in range(NBUF):
                    start_idx((g + 1) * NBUF + b, nslot + b)
        for b in range(NBUF):
            wait_gather(b)
            start_scatter(b, pslot + b)

    for b in range(NBUF):
        start_idx(b, b)
    for b in range(NBUF):
        start_idx(NBUF + b, NBUF + b)

    def outer(g2, carry):
        emit_round(2 * g2, g2, 0)
        emit_round(2 * g2 + 1, g2, 1)
        return carry

    lax.fori_loop(0, NROUND // 2, outer, 0)
    for b in range(NBUF):
        wait_scatter(b)
    plsc.subcore_barrier()
    pltpu.sync_copy(acc.at[pl.ds(base, ROWS_PER_SUB)],
                    out_hbm.at[c, pl.ds(base, ROWS_PER_SUB)])


@functools.cache
def _sc_segsum():
    return pl.kernel(
        _sc_body,
        out_type=jax.ShapeDtypeStruct((2, N_ACC, H), jnp.float32),
        mesh=plsc.VectorSubcoreMesh(core_axis_name="c", subcore_axis_name="s"),
        scratch_types=[
            pltpu.VMEM((2 * NBUF, CHUNK), jnp.int32),
            pltpu.VMEM((2 * NBUF, CHUNK), jnp.int32),
            pltpu.VMEM((NBUF, CHUNK, H), jnp.float32),
            pltpu.VMEM_SHARED((N_ACC, H), jnp.float32),
            pltpu.SemaphoreType.DMA((2 * NBUF,)),
            pltpu.SemaphoreType.DMA((NBUF,)),
            pltpu.SemaphoreType.DMA((NBUF,)),
        ],
    )


def _tcpost_body(raw_ref, deg_ref, b_ref, o_ref):
    cat = jnp.concatenate([raw_ref[0], raw_ref[1]], axis=1)
    o_ref[...] = cat / deg_ref[...] + b_ref[...]


def _tc_post(raw, deg, b):
    return pl.pallas_call(
        _tcpost_body,
        grid=(NBLK,),
        in_specs=[
            pl.BlockSpec((2, ROWS_BLK, H), lambda i: (0, i, 0)),
            pl.BlockSpec((ROWS_BLK, 1), lambda i: (i, 0)),
            pl.BlockSpec((1, D), lambda i: (0, 0)),
        ],
        out_specs=pl.BlockSpec((ROWS_BLK, D), lambda i: (i, 0)),
        out_shape=jax.ShapeDtypeStruct((N, D), jnp.float32),
    )(raw, deg, b)


def kernel(node_features, edge_index, deg, numBBs, W, b,
           gamma_bb, beta_bb, gamma_rel, beta_rel):
    nb = jnp.asarray(numBBs, jnp.int32).reshape(1, 1)
    gm = jnp.kron(jnp.eye(GROUPS, dtype=jnp.float32),
                  jnp.full((GSZ, GSZ), 1.0 / GSZ, jnp.float32))
    sup = _tc_support(nb, node_features, W, gm,
                      gamma_bb.reshape(1, D), beta_bb.reshape(1, D),
                      gamma_rel.reshape(1, D), beta_rel.reshape(1, D))
    sup = sup.reshape(2 * N, H)
    row = edge_index[0]
    col = edge_index[1]
    pad = E_PAD - E
    rowp = jnp.concatenate([row, jnp.full((pad,), N, jnp.int32)])
    colp = jnp.concatenate([col, jnp.zeros((pad,), jnp.int32)])
    if _DIAG == "seqgather":
        colp = jnp.arange(E_PAD, dtype=jnp.int32) % N
    col2 = jnp.stack([colp, colp + N])
    zeros = jnp.zeros((ROWS_PER_SUB, H), jnp.float32)
    raw = _sc_segsum()(sup, col2, rowp, zeros)
    return _tc_post(raw, deg.reshape(N, 1), b.reshape(1, D))


# speedup vs baseline: 4.0157x; 1.0152x over previous
import functools

import jax
import jax.numpy as jnp
from jax import lax
from jax.experimental import pallas as pl
from jax.experimental.pallas import tpu as pltpu
from jax.experimental.pallas import tpu_sc as plsc

N = 10000
E = 160000
D = 256
H = D // 2
GROUPS = 4
GSZ = D // GROUPS
ROWS_BLK = 400
NBLK = N // ROWS_BLK
RB_PRE = 1000
NBLK_PRE = N // RB_PRE
CHUNK = 64
N_SUBCORES = 16
ROWS_PER_SUB = 640
N_ACC = ROWS_PER_SUB * N_SUBCORES
_DIAG = ""
NBUF = 5
NROUND = 32
CHUNKS_PER_SUB = NBUF * NROUND
E_PAD = CHUNK * N_SUBCORES * CHUNKS_PER_SUB


def _tcpre_body(nb_ref, x_ref, w_ref, gm_ref, gbb_ref, bbb_ref, grel_ref,
                brel_ref, o_ref):
    i = pl.program_id(0)
    x = x_ref[...]
    gm = gm_ref[...]
    m = jnp.dot(x, gm, preferred_element_type=jnp.float32)
    ex2 = jnp.dot(x * x, gm, preferred_element_type=jnp.float32)
    xn = (x - m) * lax.rsqrt(ex2 - m * m + 1e-5)
    rows = RB_PRE * i + lax.broadcasted_iota(jnp.int32, (RB_PRE, 1), 0)
    is_bb = rows < nb_ref[0, 0]
    gamma = jnp.where(is_bb, gbb_ref[...], grel_ref[...])
    beta = jnp.where(is_bb, bbb_ref[...], brel_ref[...])
    xa = jnp.maximum(xn * gamma + beta, 0.0)
    o = jnp.dot(xa, w_ref[...], preferred_element_type=jnp.float32)
    o_ref[0] = o[:, :H]
    o_ref[1] = o[:, H:]


def _tc_support(nb, x, w, gm, gbb, bbb, grel, brel):
    return pl.pallas_call(
        _tcpre_body,
        grid=(NBLK_PRE,),
        in_specs=[
            pl.BlockSpec(memory_space=pltpu.SMEM),
            pl.BlockSpec((RB_PRE, D), lambda i: (i, 0)),
            pl.BlockSpec((D, D), lambda i: (0, 0)),
            pl.BlockSpec((D, D), lambda i: (0, 0)),
            pl.BlockSpec((1, D), lambda i: (0, 0)),
            pl.BlockSpec((1, D), lambda i: (0, 0)),
            pl.BlockSpec((1, D), lambda i: (0, 0)),
            pl.BlockSpec((1, D), lambda i: (0, 0)),
        ],
        out_specs=pl.BlockSpec((2, RB_PRE, H), lambda i: (0, i, 0)),
        out_shape=jax.ShapeDtypeStruct((2, N, H), jnp.float32),
    )(nb, x, w, gm, gbb, bbb, grel, brel)


def _sc_body(sup_hbm, col2_hbm, row_hbm, zeros_hbm, out_hbm,
             colv, rowv, gbuf, acc, semi, semg, sems):
    c = lax.axis_index("c")
    s = lax.axis_index("s")
    base = ROWS_PER_SUB * s
    pltpu.sync_copy(zeros_hbm, acc.at[pl.ds(base, ROWS_PER_SUB)])
    plsc.subcore_barrier()

    def start_idx(k, slot):
        off = CHUNK * (s + N_SUBCORES * k)
        pltpu.async_copy(col2_hbm.at[c, pl.ds(off, CHUNK)], colv.at[slot],
                         semi.at[slot])
        pltpu.async_copy(row_hbm.at[pl.ds(off, CHUNK)], rowv.at[slot],
                         semi.at[slot])

    def wait_idx(slot):
        pltpu.make_async_copy(col2_hbm.at[c, pl.ds(0, CHUNK)], colv.at[slot],
                              semi.at[slot]).wait()
        pltpu.make_async_copy(row_hbm.at[pl.ds(0, CHUNK)], rowv.at[slot],
                              semi.at[slot]).wait()

    def start_gather(b, slot):
        if _DIAG != "nogather":
            pltpu.async_copy(sup_hbm.at[colv.at[slot]], gbuf.at[b], semg.at[b])

    def wait_gather(b):
        if _DIAG != "nogather":
            pltpu.make_async_copy(sup_hbm.at[pl.ds(0, CHUNK)], gbuf.at[b],
                                  semg.at[b]).wait()

    def start_scatter(b, slot):
        if _DIAG != "noscatter":
            pltpu.async_copy(gbuf.at[b], acc.at[rowv.at[slot]], sems.at[b],
                             add=True)

    def wait_scatter(b):
        if _DIAG != "noscatter":
            pltpu.make_async_copy(sup_hbm.at[pl.ds(0, CHUNK)], gbuf.at[b],
                                  sems.at[b]).wait()

    def emit_round(g, g2, parity):
        pslot = parity * NBUF
        nslot = (1 - parity) * NBUF
        for b in range(NBUF):
            if parity == 0:
                @pl.when(g2 > 0)
                def _():
                    wait_scatter(b)
            else:
                wait_scatter(b)
            wait_idx(pslot + b)
            start_gather(b, pslot + b)
        if parity == 0:
            @pl.when(g2 > 0)
            def _():
                for b in range(NBUF):
                    start_idx((g + 1) * NBUF + b, nslot + b)
        else:
            @pl.when(g2 < NROUND // 2 - 1)
            def _():
                for b in range(NBUF):
                    start_idx((g + 1) * NBUF + b, nslot + b)
        for b in range(NBUF):
            wait_gather(b)
            start_scatter(b, pslot + b)

    for b in range(NBUF):
        start_idx(b, b)
    for b in range(NBUF):
        start_idx(NBUF + b, NBUF + b)

    def outer(g2, carry):
        emit_round(2 * g2, g2, 0)
        emit_round(2 * g2 + 1, g2, 1)
        return carry

    lax.fori_loop(0, NROUND // 2, outer, 0)
    for b in range(NBUF):
        wait_scatter(b)
    plsc.subcore_barrier()
    pltpu.sync_copy(acc.at[pl.ds(base, ROWS_PER_SUB)],
                    out_hbm.at[c, pl.ds(base, ROWS_PER_SUB)])


@functools.cache
def _sc_segsum():
    return pl.kernel(
        _sc_body,
        out_type=jax.ShapeDtypeStruct((2, N_ACC, H), jnp.float32),
        mesh=plsc.VectorSubcoreMesh(core_axis_name="c", subcore_axis_name="s"),
        scratch_types=[
            pltpu.VMEM((2 * NBUF, CHUNK), jnp.int32),
            pltpu.VMEM((2 * NBUF, CHUNK), jnp.int32),
            pltpu.VMEM((NBUF, CHUNK, H), jnp.float32),
            pltpu.VMEM_SHARED((N_ACC, H), jnp.float32),
            pltpu.SemaphoreType.DMA((2 * NBUF,)),
            pltpu.SemaphoreType.DMA((NBUF,)),
            pltpu.SemaphoreType.DMA((NBUF,)),
        ],
    )


def _tcpost_body(raw_ref, deg_ref, b_ref, o_ref):
    cat = jnp.concatenate([raw_ref[0], raw_ref[1]], axis=1)
    o_ref[...] = cat / deg_ref[...] + b_ref[...]


def _tc_post(raw, deg, b):
    return pl.pallas_call(
        _tcpost_body,
        grid=(NBLK,),
        in_specs=[
            pl.BlockSpec((2, ROWS_BLK, H), lambda i: (0, i, 0)),
            pl.BlockSpec((ROWS_BLK, 1), lambda i: (i, 0)),
            pl.BlockSpec((1, D), lambda i: (0, 0)),
        ],
        out_specs=pl.BlockSpec((ROWS_BLK, D), lambda i: (i, 0)),
        out_shape=jax.ShapeDtypeStruct((N, D), jnp.float32),
    )(raw, deg, b)


def kernel(node_features, edge_index, deg, numBBs, W, b,
           gamma_bb, beta_bb, gamma_rel, beta_rel):
    nb = jnp.asarray(numBBs, jnp.int32).reshape(1, 1)
    gm = jnp.kron(jnp.eye(GROUPS, dtype=jnp.float32),
                  jnp.full((GSZ, GSZ), 1.0 / GSZ, jnp.float32))
    sup = _tc_support(nb, node_features, W, gm,
                      gamma_bb.reshape(1, D), beta_bb.reshape(1, D),
                      gamma_rel.reshape(1, D), beta_rel.reshape(1, D))
    sup = sup.reshape(2 * N, H)
    row = edge_index[0]
    col = edge_index[1]
    pad = E_PAD - E
    rowp = jnp.concatenate([row, jnp.full((pad,), N, jnp.int32)])
    colp = jnp.concatenate([col, jnp.zeros((pad,), jnp.int32)])
    if _DIAG == "seqgather":
        colp = jnp.arange(E_PAD, dtype=jnp.int32) % N
    col2 = jnp.stack([colp, colp + N])
    zeros = jnp.zeros((ROWS_PER_SUB, H), jnp.float32)
    raw = _sc_segsum()(sup, col2, rowp, zeros)
    return _tc_post(raw, deg.reshape(N, 1), b.reshape(1, D))
